# baseline (speedup 1.0000x reference)
import functools

import jax
import jax.numpy as jnp
from jax import lax
from jax.experimental import pallas as pl
from jax.experimental.pallas import tpu as pltpu
from jax.experimental.pallas import tpu_sc as plsc

N = 10000
E = 320000
D = 128
H = 64

NSUB = 16
CH = 80
NACC = 10240
RPS = NACC // NSUB

NB = 1000
EB = 6400

E_A = 192000
E_B = E - E_A


def _prep_body(nf, wes, bes, wed, bed, wnd, bnd, wns, bns_, a_o, b_o, nd_o,
               ns_o):
    x = nf[...]
    a = jnp.dot(x, wes[...], preferred_element_type=jnp.float32) + bes[...]
    b = jnp.dot(x, wed[...], preferred_element_type=jnp.float32) + bed[...]
    c = jnp.dot(x, wnd[...], preferred_element_type=jnp.float32) + bnd[...]
    ns = jnp.dot(x, wns[...], preferred_element_type=jnp.float32) + bns_[...]
    a_o[0] = a[:, :H]
    a_o[1] = a[:, H:]
    b_o[0] = b[:, :H]
    b_o[1] = b[:, H:]
    nd_o[...] = c
    ns_o[...] = ns


def _prep(nf, wes, bes, wed, bed, wnd, bnd, wns, bns_):
    wspec = pl.BlockSpec((D, D), lambda i: (0, 0))
    bspec = pl.BlockSpec((1, D), lambda i: (0, 0))
    return pl.pallas_call(
        _prep_body,
        grid=(N // NB,),
        in_specs=[
            pl.BlockSpec((NB, D), lambda i: (i, 0)),
            wspec, bspec, wspec, bspec, wspec, bspec, wspec, bspec,
        ],
        out_specs=[
            pl.BlockSpec((2, NB, H), lambda i: (0, i, 0)),
            pl.BlockSpec((2, NB, H), lambda i: (0, i, 0)),
            pl.BlockSpec((NB, D), lambda i: (i, 0)),
            pl.BlockSpec((NB, D), lambda i: (i, 0)),
        ],
        out_shape=[
            jax.ShapeDtypeStruct((2, N, H), jnp.float32),
            jax.ShapeDtypeStruct((2, N, H), jnp.float32),
            jax.ShapeDtypeStruct((N, D), jnp.float32),
            jax.ShapeDtypeStruct((N, D), jnp.float32),
        ],
    )(nf, wes, bes, wed, bed, wnd, bnd, wns, bns_)


def _edge_mm_body(ef, w, b, o):
    y = jnp.dot(ef[...], w[...], preferred_element_type=jnp.float32) + b[...]
    o[0] = y[:, :H]
    o[1] = y[:, H:]


def _edge_mm(ef, w, b, eh, off):
    return pl.pallas_call(
        _edge_mm_body,
        grid=(eh // EB,),
        in_specs=[
            pl.BlockSpec((EB, D), lambda i: (i + off, 0)),
            pl.BlockSpec((D, D), lambda i: (0, 0)),
            pl.BlockSpec((1, D), lambda i: (0, 0)),
        ],
        out_specs=pl.BlockSpec((2, EB, H), lambda i: (0, i, 0)),
        out_shape=jax.ShapeDtypeStruct((2, eh, H), jnp.float32),
    )(ef, w, b)


def _make_sc_body(eh, e0):
    eps = eh // NSUB
    nchunk = eps // CH

    def _sc_body(a2, b2, efw2, src, dst,
                 upd_o, s2_o, bns_o,
                 src_v0, src_v1, dst_v0, dst_v1, dsta_v0, dsta_v1,
                 dst_s0, dst_s1,
                 a_buf0, a_buf1, b_buf0, b_buf1, efw_buf0, efw_buf1,
                 upd_buf0, upd_buf1, gate_buf0, gate_buf1,
                 bn_buf, red_buf, s2acc, stage,
                 idx_sem0, idx_sem1, in_sem0, in_sem1, out_sem0, out_sem1):
        c = lax.axis_index("c")
        s = lax.axis_index("s")
        r0 = s * RPS
        zv = jnp.zeros((16,), jnp.float32)

        def zrow(r, carry):
            for j in range(4):
                upd_buf0[r, pl.ds(j * 16, 16)] = zv
            return carry

        lax.fori_loop(0, CH, zrow, 0)
        for m in range(RPS // CH):
            pltpu.async_copy(upd_buf0, s2acc.at[pl.ds(r0 + m * CH, CH)],
                             in_sem0)
        for m in range(RPS // CH):
            pltpu.make_async_copy(upd_buf0, s2acc.at[pl.ds(r0, CH)],
                                  in_sem0).wait()
        plsc.subcore_barrier()

        cn = c * N
        ebase = e0 + s * eps
        lbase = s * eps
        src_v = (src_v0, src_v1)
        dst_v = (dst_v0, dst_v1)
        dsta_v = (dsta_v0, dsta_v1)
        dst_s = (dst_s0, dst_s1)
        a_buf = (a_buf0, a_buf1)
        b_buf = (b_buf0, b_buf1)
        efw_buf = (efw_buf0, efw_buf1)
        upd_buf = (upd_buf0, upd_buf1)
        gate_buf = (gate_buf0, gate_buf1)
        idx_sem = (idx_sem0, idx_sem1)
        in_sem = (in_sem0, in_sem1)
        out_sem = (out_sem0, out_sem1)

        def issue_idx(j, p):
            jb = ebase + j * CH
            pltpu.async_copy(src.at[pl.ds(jb, CH)], src_v[p], idx_sem[p])
            pltpu.async_copy(dst.at[pl.ds(jb, CH)], dst_v[p], idx_sem[p])

        def wait_idx(p):
            pltpu.make_async_copy(src.at[pl.ds(0, CH)], src_v[p],
                                  idx_sem[p]).wait()
            pltpu.make_async_copy(dst.at[pl.ds(0, CH)], dst_v[p],
                                  idx_sem[p]).wait()

        def adjust(p):
            for k in range(CH // 16):
                sl = pl.ds(k * 16, 16)
                src_v[p][sl] = src_v[p][sl] + cn
                dsta_v[p][sl] = dst_v[p][sl] + cn

        def issue_in(j, p):
            pltpu.async_copy(a2.at[src_v[p]], a_buf[p], in_sem[p])
            pltpu.async_copy(b2.at[dsta_v[p]], b_buf[p], in_sem[p])
            pltpu.async_copy(efw2.at[pl.ds(c * eh + lbase + j * CH, CH)],
                             efw_buf[p], in_sem[p])

        def wait_in(p):
            pltpu.make_async_copy(a2.at[src_v[p]], a_buf[p], in_sem[p]).wait()
            pltpu.make_async_copy(b2.at[dsta_v[p]], b_buf[p], in_sem[p]).wait()
            pltpu.make_async_copy(efw2.at[pl.ds(0, CH)], efw_buf[p],
                                  in_sem[p]).wait()

        def drain_out(p):
            pltpu.make_async_copy(upd_buf[p], upd_o.at[pl.ds(0, CH)],
                                  out_sem[p]).wait()

        def compute(p, bn):
            ab, bb, eb, ub, gb = a_buf[p], b_buf[p], efw_buf[p], upd_buf[p], \
                gate_buf[p]

            def row(r, bn_c):
                out = list(bn_c)
                for j in range(4):
                    sl = pl.ds(j * 16, 16)
                    u = eb[r, sl] + ab[r, sl] + bb[r, sl]
                    ub[r, sl] = u
                    g = u / (1.0 + jnp.exp(-u))
                    gb[r, sl] = g
                    out[j] = bn_c[j] + u
                    out[4 + j] = bn_c[4 + j] + u * u
                return tuple(out)

            return lax.fori_loop(0, CH, row, bn)

        def sub(t, j, p, q, guard_next, bn):
            @pl.when(t >= 1)
            def _():
                drain_out(p)

            wait_in(p)
            for k in range(CH // 16):
                sl = pl.ds(k * 16, 16)
                dst_s[p][sl] = dst_v[p][sl]

            @pl.when(t <= nchunk // 2 - 2)
            def _():
                issue_idx(j + 2, p)

            def start_next():
                wait_idx(q)
                adjust(q)
                issue_in(j + 1, q)

            if guard_next:
                @pl.when(t <= nchunk // 2 - 2)
                def _():
                    start_next()
            else:
                start_next()

            bn = compute(p, bn)
            pltpu.async_copy(upd_buf[p],
                             upd_o.at[pl.ds(c * eh + lbase + j * CH, CH)],
                             out_sem[p])
            pltpu.sync_copy(gate_buf[p], s2acc.at[dst_s[p]], add=True)
            return bn

        issue_idx(0, 0)
        issue_idx(1, 1)
        wait_idx(0)
        adjust(0)
        issue_in(0, 0)

        zero16 = jnp.zeros((16,), jnp.float32)

        def body(t, bn):
            bn = sub(t, 2 * t, 0, 1, False, bn)
            bn = sub(t, 2 * t + 1, 1, 0, True, bn)
            return bn

        bn = lax.fori_loop(0, nchunk // 2, body, (zero16,) * 8)
        drain_out(0)
        drain_out(1)

        for j in range(4):
            sl = pl.ds(j * 16, 16)
            bn_buf[0, sl] = bn[j]
            bn_buf[1, sl] = bn[4 + j]
            for r in range(2, 8):
                bn_buf[r, sl] = zero16
        pltpu.sync_copy(bn_buf, stage.at[s])
        plsc.subcore_barrier()

        ca = c * NACC
        pltpu.sync_copy(s2acc.at[pl.ds(r0, RPS)], s2_o.at[pl.ds(ca + r0, RPS)])

        @pl.when(s == 0)
        def _():
            pltpu.sync_copy(stage, red_buf)
            for j in range(8):
                row_i = j // 4
                sl = pl.ds((j % 4) * 16, 16)
                acc = red_buf[0, row_i, sl]
                for t in range(1, NSUB):
                    acc = acc + red_buf[t, row_i, sl]
                bn_buf[row_i, sl] = acc
            pltpu.sync_copy(bn_buf, bns_o.at[c])

    return _sc_body


def _sc_half(a2, b2, efw2, src, dst, eh, e0):
    mesh = plsc.VectorSubcoreMesh(core_axis_name="c", subcore_axis_name="s")
    fn = pl.kernel(
        _make_sc_body(eh, e0),
        out_type=[
            jax.ShapeDtypeStruct((2 * eh, H), jnp.float32),
            jax.ShapeDtypeStruct((2 * NACC, H), jnp.float32),
            jax.ShapeDtypeStruct((2, 8, H), jnp.float32),
        ],
        mesh=mesh,
        scratch_types=(
            [pltpu.VMEM((CH,), jnp.int32)] * 8
            + [pltpu.VMEM((CH, H), jnp.float32)] * 10
            + [
                pltpu.VMEM((8, H), jnp.float32),
                pltpu.VMEM((NSUB, 8, H), jnp.float32),
                pltpu.VMEM_SHARED((NACC, H), jnp.float32),
                pltpu.VMEM_SHARED((NSUB, 8, H), jnp.float32),
            ]
            + [pltpu.SemaphoreType.DMA] * 6
        ),
        compiler_params=pltpu.CompilerParams(use_tc_tiling_on_sc=False),
    )
    return fn(a2, b2, efw2, src, dst)


def _silu(x):
    return x / (1.0 + jnp.exp(-x))


def _edge_post_body(upd, ef, bns_a, bns_b, gamma, beta, *rest):
    o = rest[-1]
    u = jnp.concatenate([upd[0], upd[1]], axis=1)
    st = bns_a[...] + bns_b[...]
    mean = jnp.concatenate([st[0, 0:1, :], st[1, 0:1, :]], axis=1) * (1.0 / E)
    msq = jnp.concatenate([st[0, 1:2, :], st[1, 1:2, :]], axis=1) * (1.0 / E)
    var = msq - mean * mean
    xh = (u - mean) / jnp.sqrt(var + 1e-5) * gamma[...] + beta[...]
    o[...] = _silu(xh) + ef[...]


def _edge_post(upd3, ef, bns_a, bns_b, gamma, beta, eh, off, prev=None):
    in_specs = [
        pl.BlockSpec((2, EB, H), lambda i: (0, i, 0)),
        pl.BlockSpec((EB, D), lambda i: (i + off, 0)),
        pl.BlockSpec((2, 8, H), lambda i: (0, 0, 0)),
        pl.BlockSpec((2, 8, H), lambda i: (0, 0, 0)),
        pl.BlockSpec((1, D), lambda i: (0, 0)),
        pl.BlockSpec((1, D), lambda i: (0, 0)),
    ]
    args = [upd3, ef, bns_a, bns_b, gamma, beta]
    aliases = {}
    if prev is not None:
        in_specs.append(pl.BlockSpec(memory_space=pltpu.MemorySpace.HBM))
        args.append(prev)
        aliases = {6: 0}
    return pl.pallas_call(
        _edge_post_body,
        grid=(eh // EB,),
        in_specs=in_specs,
        out_specs=pl.BlockSpec((EB, D), lambda i: (i + off, 0)),
        out_shape=jax.ShapeDtypeStruct((E, D), jnp.float32),
        input_output_aliases=aliases,
    )(*args)


def _node_post_body(ns, nd, s2a, s2b, nf, gamma, beta, o, ssum, ssq):
    p = pl.program_id(0)
    i = pl.program_id(1)
    g2 = jnp.concatenate([s2a[0] + s2b[0], s2a[1] + s2b[1]], axis=1)
    upd = ns[...] + nd[...] * (g2 / (g2 + 1e-6))

    @pl.when(p == 0)
    def _():
        @pl.when(i == 0)
        def _():
            ssum[...] = jnp.zeros_like(ssum)
            ssq[...] = jnp.zeros_like(ssq)

        ssum[...] += jnp.sum(upd, axis=0, keepdims=True)
        ssq[...] += jnp.sum(upd * upd, axis=0, keepdims=True)
        o[...] = jnp.zeros_like(o)

    @pl.when(p == 1)
    def _():
        mean = ssum[...] * (1.0 / N)
        var = ssq[...] * (1.0 / N) - mean * mean
        xh = (upd - mean) / jnp.sqrt(var + 1e-5) * gamma[...] + beta[...]
        o[...] = _silu(xh) + nf[...]


def _node_post(ns, nd, s2a, s2b, nf, gamma, beta):
    return pl.pallas_call(
        _node_post_body,
        grid=(2, N // NB),
        in_specs=[
            pl.BlockSpec((NB, D), lambda p, i: (i, 0)),
            pl.BlockSpec((NB, D), lambda p, i: (i, 0)),
            pl.BlockSpec((2, NB, H), lambda p, i: (0, i, 0)),
            pl.BlockSpec((2, NB, H), lambda p, i: (0, i, 0)),
            pl.BlockSpec((NB, D), lambda p, i: (i, 0)),
            pl.BlockSpec((1, D), lambda p, i: (0, 0)),
            pl.BlockSpec((1, D), lambda p, i: (0, 0)),
        ],
        out_specs=pl.BlockSpec((NB, D), lambda p, i: (i, 0)),
        out_shape=jax.ShapeDtypeStruct((N, D), jnp.float32),
        scratch_shapes=[
            pltpu.VMEM((1, D), jnp.float32),
            pltpu.VMEM((1, D), jnp.float32),
        ],
    )(ns, nd, s2a, s2b, nf, gamma, beta)


def kernel(node_feats, edge_feats, W_esrc, b_esrc, W_edst, b_edst, W_eedge,
           b_eedge, W_nsrc, b_nsrc, W_ndst, b_ndst, gamma_e, beta_e, gamma_n,
           beta_n, edge_index):
    src = edge_index[0]
    dst = edge_index[1]
    a3, b3, nd, ns = _prep(node_feats,
                           W_esrc, b_esrc.reshape(1, D),
                           W_edst, b_edst.reshape(1, D),
                           W_ndst, b_ndst.reshape(1, D),
                           W_nsrc, b_nsrc.reshape(1, D))
    a2 = a3.reshape(2 * N, H)
    b2 = b3.reshape(2 * N, H)
    ge = gamma_e.reshape(1, D)
    be = beta_e.reshape(1, D)

    efw_a = _edge_mm(edge_feats, W_eedge, b_eedge.reshape(1, D), E_A, 0)
    upd_a, s2a, bns_a = _sc_half(a2, b2, efw_a.reshape(2 * E_A, H),
                                 src, dst, E_A, 0)
    efw_b = _edge_mm(edge_feats, W_eedge, b_eedge.reshape(1, D), E_B,
                     E_A // EB)
    upd_b, s2b, bns_b = _sc_half(a2, b2, efw_b.reshape(2 * E_B, H),
                                 src, dst, E_B, E_A)
    out_a = _edge_post(upd_a.reshape(2, E_A, H), edge_feats, bns_a, bns_b,
                       ge, be, E_A, 0)
    edge_out = _edge_post(upd_b.reshape(2, E_B, H), edge_feats, bns_a, bns_b,
                          ge, be, E_B, E_A // EB, prev=out_a)
    node_out = _node_post(ns, nd, s2a.reshape(2, NACC, H),
                          s2b.reshape(2, NACC, H), node_feats,
                          gamma_n.reshape(1, D), beta_n.reshape(1, D))
    return (node_out, edge_out)


# speedup vs baseline: 1.0883x; 1.0883x over previous
import functools

import jax
import jax.numpy as jnp
from jax import lax
from jax.experimental import pallas as pl
from jax.experimental.pallas import tpu as pltpu
from jax.experimental.pallas import tpu_sc as plsc

N = 10000
E = 320000
D = 128
H = 64

NSUB = 16
CH = 80
NACC = 10240
RPS = NACC // NSUB

NB = 1000
EB = 6400

E_A = 166400
E_B = E - E_A


def _prep_body(nf, wes, bes, wed, bed, wnd, bnd, wns, bns_, a_o, b_o, nd_o,
               ns_o):
    x = nf[...]
    a = jnp.dot(x, wes[...], preferred_element_type=jnp.float32) + bes[...]
    b = jnp.dot(x, wed[...], preferred_element_type=jnp.float32) + bed[...]
    c = jnp.dot(x, wnd[...], preferred_element_type=jnp.float32) + bnd[...]
    ns = jnp.dot(x, wns[...], preferred_element_type=jnp.float32) + bns_[...]
    a_o[0] = a[:, :H]
    a_o[1] = a[:, H:]
    b_o[0] = b[:, :H]
    b_o[1] = b[:, H:]
    nd_o[...] = c
    ns_o[...] = ns


def _prep(nf, wes, bes, wed, bed, wnd, bnd, wns, bns_):
    wspec = pl.BlockSpec((D, D), lambda i: (0, 0))
    bspec = pl.BlockSpec((1, D), lambda i: (0, 0))
    return pl.pallas_call(
        _prep_body,
        grid=(N // NB,),
        in_specs=[
            pl.BlockSpec((NB, D), lambda i: (i, 0)),
            wspec, bspec, wspec, bspec, wspec, bspec, wspec, bspec,
        ],
        out_specs=[
            pl.BlockSpec((2, NB, H), lambda i: (0, i, 0)),
            pl.BlockSpec((2, NB, H), lambda i: (0, i, 0)),
            pl.BlockSpec((NB, D), lambda i: (i, 0)),
            pl.BlockSpec((NB, D), lambda i: (i, 0)),
        ],
        out_shape=[
            jax.ShapeDtypeStruct((2, N, H), jnp.float32),
            jax.ShapeDtypeStruct((2, N, H), jnp.float32),
            jax.ShapeDtypeStruct((N, D), jnp.float32),
            jax.ShapeDtypeStruct((N, D), jnp.float32),
        ],
    )(nf, wes, bes, wed, bed, wnd, bnd, wns, bns_)


def _edge_mm_body(ef, w, b, o):
    y = jnp.dot(ef[...], w[...], preferred_element_type=jnp.float32) + b[...]
    o[0] = y[:, :H]
    o[1] = y[:, H:]


def _edge_mm(ef, w, b, eh, off):
    return pl.pallas_call(
        _edge_mm_body,
        grid=(eh // EB,),
        in_specs=[
            pl.BlockSpec((EB, D), lambda i: (i + off, 0)),
            pl.BlockSpec((D, D), lambda i: (0, 0)),
            pl.BlockSpec((1, D), lambda i: (0, 0)),
        ],
        out_specs=pl.BlockSpec((2, EB, H), lambda i: (0, i, 0)),
        out_shape=jax.ShapeDtypeStruct((2, eh, H), jnp.float32),
    )(ef, w, b)


def _make_sc_body(eh, e0):
    eps = eh // NSUB
    nchunk = eps // CH

    def _sc_body(a2, b2, efw2, src, dst,
                 upd_o, s2_o, bns_o,
                 src_v0, src_v1, dst_v0, dst_v1, dsta_v0, dsta_v1,
                 dst_s0, dst_s1,
                 a_buf0, a_buf1, b_buf0, b_buf1, efw_buf0, efw_buf1,
                 upd_buf0, upd_buf1, gate_buf0, gate_buf1,
                 bn_buf, red_buf, s2acc, stage,
                 idx_sem0, idx_sem1, in_sem0, in_sem1, out_sem0, out_sem1):
        c = lax.axis_index("c")
        s = lax.axis_index("s")
        r0 = s * RPS
        zv = jnp.zeros((16,), jnp.float32)

        def zrow(r, carry):
            for j in range(4):
                upd_buf0[r, pl.ds(j * 16, 16)] = zv
            return carry

        lax.fori_loop(0, CH, zrow, 0)
        for m in range(RPS // CH):
            pltpu.async_copy(upd_buf0, s2acc.at[pl.ds(r0 + m * CH, CH)],
                             in_sem0)
        for m in range(RPS // CH):
            pltpu.make_async_copy(upd_buf0, s2acc.at[pl.ds(r0, CH)],
                                  in_sem0).wait()
        plsc.subcore_barrier()

        cn = c * N
        ebase = e0 + s * eps
        lbase = s * eps
        src_v = (src_v0, src_v1)
        dst_v = (dst_v0, dst_v1)
        dsta_v = (dsta_v0, dsta_v1)
        dst_s = (dst_s0, dst_s1)
        a_buf = (a_buf0, a_buf1)
        b_buf = (b_buf0, b_buf1)
        efw_buf = (efw_buf0, efw_buf1)
        upd_buf = (upd_buf0, upd_buf1)
        gate_buf = (gate_buf0, gate_buf1)
        idx_sem = (idx_sem0, idx_sem1)
        in_sem = (in_sem0, in_sem1)
        out_sem = (out_sem0, out_sem1)

        def issue_idx(j, p):
            jb = ebase + j * CH
            pltpu.async_copy(src.at[pl.ds(jb, CH)], src_v[p], idx_sem[p])
            pltpu.async_copy(dst.at[pl.ds(jb, CH)], dst_v[p], idx_sem[p])

        def wait_idx(p):
            pltpu.make_async_copy(src.at[pl.ds(0, CH)], src_v[p],
                                  idx_sem[p]).wait()
            pltpu.make_async_copy(dst.at[pl.ds(0, CH)], dst_v[p],
                                  idx_sem[p]).wait()

        def adjust(p):
            for k in range(CH // 16):
                sl = pl.ds(k * 16, 16)
                src_v[p][sl] = src_v[p][sl] + cn
                dsta_v[p][sl] = dst_v[p][sl] + cn

        def issue_in(j, p):
            pltpu.async_copy(a2.at[src_v[p]], a_buf[p], in_sem[p])
            pltpu.async_copy(b2.at[dsta_v[p]], b_buf[p], in_sem[p])
            pltpu.async_copy(efw2.at[pl.ds(c * eh + lbase + j * CH, CH)],
                             efw_buf[p], in_sem[p])

        def wait_in(p):
            pltpu.make_async_copy(a2.at[src_v[p]], a_buf[p], in_sem[p]).wait()
            pltpu.make_async_copy(b2.at[dsta_v[p]], b_buf[p], in_sem[p]).wait()
            pltpu.make_async_copy(efw2.at[pl.ds(0, CH)], efw_buf[p],
                                  in_sem[p]).wait()

        def drain_out(p):
            pltpu.make_async_copy(upd_buf[p], upd_o.at[pl.ds(0, CH)],
                                  out_sem[p]).wait()

        def compute(p, bn):
            ab, bb, eb, ub, gb = a_buf[p], b_buf[p], efw_buf[p], upd_buf[p], \
                gate_buf[p]

            def row(r, bn_c):
                out = list(bn_c)
                for j in range(4):
                    sl = pl.ds(j * 16, 16)
                    u = eb[r, sl] + ab[r, sl] + bb[r, sl]
                    ub[r, sl] = u
                    g = u / (1.0 + jnp.exp(-u))
                    gb[r, sl] = g
                    out[j] = bn_c[j] + u
                    out[4 + j] = bn_c[4 + j] + u * u
                return tuple(out)

            return lax.fori_loop(0, CH, row, bn)

        def sub(t, j, p, q, guard_next, bn):
            @pl.when(t >= 1)
            def _():
                drain_out(p)

            wait_in(p)
            for k in range(CH // 16):
                sl = pl.ds(k * 16, 16)
                dst_s[p][sl] = dst_v[p][sl]

            @pl.when(t <= nchunk // 2 - 2)
            def _():
                issue_idx(j + 2, p)

            def start_next():
                wait_idx(q)
                adjust(q)
                issue_in(j + 1, q)

            if guard_next:
                @pl.when(t <= nchunk // 2 - 2)
                def _():
                    start_next()
            else:
                start_next()

            bn = compute(p, bn)
            pltpu.async_copy(upd_buf[p],
                             upd_o.at[pl.ds(c * eh + lbase + j * CH, CH)],
                             out_sem[p])
            pltpu.sync_copy(gate_buf[p], s2acc.at[dst_s[p]], add=True)
            return bn

        issue_idx(0, 0)
        issue_idx(1, 1)
        wait_idx(0)
        adjust(0)
        issue_in(0, 0)

        zero16 = jnp.zeros((16,), jnp.float32)

        def body(t, bn):
            bn = sub(t, 2 * t, 0, 1, False, bn)
            bn = sub(t, 2 * t + 1, 1, 0, True, bn)
            return bn

        bn = lax.fori_loop(0, nchunk // 2, body, (zero16,) * 8)
        drain_out(0)
        drain_out(1)

        for j in range(4):
            sl = pl.ds(j * 16, 16)
            bn_buf[0, sl] = bn[j]
            bn_buf[1, sl] = bn[4 + j]
            for r in range(2, 8):
                bn_buf[r, sl] = zero16
        pltpu.sync_copy(bn_buf, stage.at[s])
        plsc.subcore_barrier()

        ca = c * NACC
        pltpu.sync_copy(s2acc.at[pl.ds(r0, RPS)], s2_o.at[pl.ds(ca + r0, RPS)])

        @pl.when(s == 0)
        def _():
            pltpu.sync_copy(stage, red_buf)
            for j in range(8):
                row_i = j // 4
                sl = pl.ds((j % 4) * 16, 16)
                acc = red_buf[0, row_i, sl]
                for t in range(1, NSUB):
                    acc = acc + red_buf[t, row_i, sl]
                bn_buf[row_i, sl] = acc
            pltpu.sync_copy(bn_buf, bns_o.at[c])

    return _sc_body


def _sc_half(a2, b2, efw2, src, dst, eh, e0):
    mesh = plsc.VectorSubcoreMesh(core_axis_name="c", subcore_axis_name="s")
    fn = pl.kernel(
        _make_sc_body(eh, e0),
        out_type=[
            jax.ShapeDtypeStruct((2 * eh, H), jnp.float32),
            jax.ShapeDtypeStruct((2 * NACC, H), jnp.float32),
            jax.ShapeDtypeStruct((2, 8, H), jnp.float32),
        ],
        mesh=mesh,
        scratch_types=(
            [pltpu.VMEM((CH,), jnp.int32)] * 8
            + [pltpu.VMEM((CH, H), jnp.float32)] * 10
            + [
                pltpu.VMEM((8, H), jnp.float32),
                pltpu.VMEM((NSUB, 8, H), jnp.float32),
                pltpu.VMEM_SHARED((NACC, H), jnp.float32),
                pltpu.VMEM_SHARED((NSUB, 8, H), jnp.float32),
            ]
            + [pltpu.SemaphoreType.DMA] * 6
        ),
        compiler_params=pltpu.CompilerParams(use_tc_tiling_on_sc=False),
    )
    return fn(a2, b2, efw2, src, dst)


def _silu(x):
    return x / (1.0 + jnp.exp(-x))


def _edge_post_body(upd, ef, bns_a, bns_b, gamma, beta, *rest):
    o = rest[-1]
    u = jnp.concatenate([upd[0], upd[1]], axis=1)
    st = bns_a[...] + bns_b[...]
    mean = jnp.concatenate([st[0, 0:1, :], st[1, 0:1, :]], axis=1) * (1.0 / E)
    msq = jnp.concatenate([st[0, 1:2, :], st[1, 1:2, :]], axis=1) * (1.0 / E)
    var = msq - mean * mean
    xh = (u - mean) / jnp.sqrt(var + 1e-5) * gamma[...] + beta[...]
    o[...] = _silu(xh) + ef[...]


def _edge_post(upd3, ef, bns_a, bns_b, gamma, beta, eh, off, prev=None):
    in_specs = [
        pl.BlockSpec((2, EB, H), lambda i: (0, i, 0)),
        pl.BlockSpec((EB, D), lambda i: (i + off, 0)),
        pl.BlockSpec((2, 8, H), lambda i: (0, 0, 0)),
        pl.BlockSpec((2, 8, H), lambda i: (0, 0, 0)),
        pl.BlockSpec((1, D), lambda i: (0, 0)),
        pl.BlockSpec((1, D), lambda i: (0, 0)),
    ]
    args = [upd3, ef, bns_a, bns_b, gamma, beta]
    aliases = {}
    if prev is not None:
        in_specs.append(pl.BlockSpec(memory_space=pltpu.MemorySpace.HBM))
        args.append(prev)
        aliases = {6: 0}
    return pl.pallas_call(
        _edge_post_body,
        grid=(eh // EB,),
        in_specs=in_specs,
        out_specs=pl.BlockSpec((EB, D), lambda i: (i + off, 0)),
        out_shape=jax.ShapeDtypeStruct((E, D), jnp.float32),
        input_output_aliases=aliases,
    )(*args)


def _node_post_body(ns, nd, s2a, s2b, nf, gamma, beta, o, ssum, ssq):
    p = pl.program_id(0)
    i = pl.program_id(1)
    g2 = jnp.concatenate([s2a[0] + s2b[0], s2a[1] + s2b[1]], axis=1)
    upd = ns[...] + nd[...] * (g2 / (g2 + 1e-6))

    @pl.when(p == 0)
    def _():
        @pl.when(i == 0)
        def _():
            ssum[...] = jnp.zeros_like(ssum)
            ssq[...] = jnp.zeros_like(ssq)

        ssum[...] += jnp.sum(upd, axis=0, keepdims=True)
        ssq[...] += jnp.sum(upd * upd, axis=0, keepdims=True)
        o[...] = jnp.zeros_like(o)

    @pl.when(p == 1)
    def _():
        mean = ssum[...] * (1.0 / N)
        var = ssq[...] * (1.0 / N) - mean * mean
        xh = (upd - mean) / jnp.sqrt(var + 1e-5) * gamma[...] + beta[...]
        o[...] = _silu(xh) + nf[...]


def _node_post(ns, nd, s2a, s2b, nf, gamma, beta):
    return pl.pallas_call(
        _node_post_body,
        grid=(2, N // NB),
        in_specs=[
            pl.BlockSpec((NB, D), lambda p, i: (i, 0)),
            pl.BlockSpec((NB, D), lambda p, i: (i, 0)),
            pl.BlockSpec((2, NB, H), lambda p, i: (0, i, 0)),
            pl.BlockSpec((2, NB, H), lambda p, i: (0, i, 0)),
            pl.BlockSpec((NB, D), lambda p, i: (i, 0)),
            pl.BlockSpec((1, D), lambda p, i: (0, 0)),
            pl.BlockSpec((1, D), lambda p, i: (0, 0)),
        ],
        out_specs=pl.BlockSpec((NB, D), lambda p, i: (i, 0)),
        out_shape=jax.ShapeDtypeStruct((N, D), jnp.float32),
        scratch_shapes=[
            pltpu.VMEM((1, D), jnp.float32),
            pltpu.VMEM((1, D), jnp.float32),
        ],
    )(ns, nd, s2a, s2b, nf, gamma, beta)


def kernel(node_feats, edge_feats, W_esrc, b_esrc, W_edst, b_edst, W_eedge,
           b_eedge, W_nsrc, b_nsrc, W_ndst, b_ndst, gamma_e, beta_e, gamma_n,
           beta_n, edge_index):
    src = edge_index[0]
    dst = edge_index[1]
    a3, b3, nd, ns = _prep(node_feats,
                           W_esrc, b_esrc.reshape(1, D),
                           W_edst, b_edst.reshape(1, D),
                           W_ndst, b_ndst.reshape(1, D),
                           W_nsrc, b_nsrc.reshape(1, D))
    a2 = a3.reshape(2 * N, H)
    b2 = b3.reshape(2 * N, H)
    ge = gamma_e.reshape(1, D)
    be = beta_e.reshape(1, D)

    efw_a = _edge_mm(edge_feats, W_eedge, b_eedge.reshape(1, D), E_A, 0)
    upd_a, s2a, bns_a = _sc_half(a2, b2, efw_a.reshape(2 * E_A, H),
                                 src, dst, E_A, 0)
    efw_b = _edge_mm(edge_feats, W_eedge, b_eedge.reshape(1, D), E_B,
                     E_A // EB)
    upd_b, s2b, bns_b = _sc_half(a2, b2, efw_b.reshape(2 * E_B, H),
                                 src, dst, E_B, E_A)
    out_a = _edge_post(upd_a.reshape(2, E_A, H), edge_feats, bns_a, bns_b,
                       ge, be, E_A, 0)
    edge_out = _edge_post(upd_b.reshape(2, E_B, H), edge_feats, bns_a, bns_b,
                          ge, be, E_B, E_A // EB, prev=out_a)
    node_out = _node_post(ns, nd, s2a.reshape(2, NACC, H),
                          s2b.reshape(2, NACC, H), node_feats,
                          gamma_n.reshape(1, D), beta_n.reshape(1, D))
    return (node_out, edge_out)


# speedup vs baseline: 1.0891x; 1.0007x over previous
import jax
import jax.numpy as jnp
from jax import lax
from jax.experimental import pallas as pl
from jax.experimental.pallas import tpu as pltpu
from jax.experimental.pallas import tpu_sc as plsc

N = 10000
E = 320000
D = 128
H = 64

NSUB = 16
CH = 80
NACC = 10240
RPS = NACC // NSUB

NB = 1000
EB = 6400

E_A = 166400
E_B = E - E_A


def _prep_body(nf, wes, bes, wed, bed, wnd, bnd, wns, bns_, a_o, b_o, nd_o,
               ns_o):
    x = nf[...]
    a = jnp.dot(x, wes[...], preferred_element_type=jnp.float32) + bes[...]
    b = jnp.dot(x, wed[...], preferred_element_type=jnp.float32) + bed[...]
    c = jnp.dot(x, wnd[...], preferred_element_type=jnp.float32) + bnd[...]
    ns = jnp.dot(x, wns[...], preferred_element_type=jnp.float32) + bns_[...]
    a_o[0] = a[:, :H]
    a_o[1] = a[:, H:]
    b_o[0] = b[:, :H]
    b_o[1] = b[:, H:]
    nd_o[...] = c
    ns_o[...] = ns


def _prep(nf, wes, bes, wed, bed, wnd, bnd, wns, bns_):
    wspec = pl.BlockSpec((D, D), lambda i: (0, 0))
    bspec = pl.BlockSpec((1, D), lambda i: (0, 0))
    return pl.pallas_call(
        _prep_body,
        grid=(N // NB,),
        in_specs=[
            pl.BlockSpec((NB, D), lambda i: (i, 0)),
            wspec, bspec, wspec, bspec, wspec, bspec, wspec, bspec,
        ],
        out_specs=[
            pl.BlockSpec((2, NB, H), lambda i: (0, i, 0)),
            pl.BlockSpec((2, NB, H), lambda i: (0, i, 0)),
            pl.BlockSpec((NB, D), lambda i: (i, 0)),
            pl.BlockSpec((NB, D), lambda i: (i, 0)),
        ],
        out_shape=[
            jax.ShapeDtypeStruct((2, N, H), jnp.float32),
            jax.ShapeDtypeStruct((2, N, H), jnp.float32),
            jax.ShapeDtypeStruct((N, D), jnp.float32),
            jax.ShapeDtypeStruct((N, D), jnp.float32),
        ],
    )(nf, wes, bes, wed, bed, wnd, bnd, wns, bns_)


def _edge_mm_body(ef, w, b, o):
    y = jnp.dot(ef[...], w[...], preferred_element_type=jnp.float32) + b[...]
    o[0] = y[:, :H]
    o[1] = y[:, H:]


def _edge_mm(ef, w, b, eh, off):
    return pl.pallas_call(
        _edge_mm_body,
        grid=(eh // EB,),
        in_specs=[
            pl.BlockSpec((EB, D), lambda i: (i + off, 0)),
            pl.BlockSpec((D, D), lambda i: (0, 0)),
            pl.BlockSpec((1, D), lambda i: (0, 0)),
        ],
        out_specs=pl.BlockSpec((2, EB, H), lambda i: (0, i, 0)),
        out_shape=jax.ShapeDtypeStruct((2, eh, H), jnp.float32),
    )(ef, w, b)


def _make_sc_body(eh, e0):
    eps = eh // NSUB
    nchunk = eps // CH

    def _sc_body(a2, b2, efw2, src, dst,
                 upd_o, s2_o, bns_o,
                 src_v0, src_v1, dst_v0, dst_v1, dsta_v0, dsta_v1,
                 dst_s0, dst_s1,
                 a_buf0, a_buf1, b_buf0, b_buf1, efw_buf0, efw_buf1,
                 upd_buf0, upd_buf1, gate_buf0, gate_buf1,
                 bn_buf, red_buf, s2acc, stage,
                 idx_sem0, idx_sem1, in_sem0, in_sem1, out_sem0, out_sem1):
        c = lax.axis_index("c")
        s = lax.axis_index("s")
        r0 = s * RPS
        zv = jnp.zeros((16,), jnp.float32)

        def zrow(r, carry):
            for j in range(4):
                upd_buf0[r, pl.ds(j * 16, 16)] = zv
            return carry

        lax.fori_loop(0, CH, zrow, 0)
        for m in range(RPS // CH):
            pltpu.async_copy(upd_buf0, s2acc.at[pl.ds(r0 + m * CH, CH)],
                             in_sem0)
        for m in range(RPS // CH):
            pltpu.make_async_copy(upd_buf0, s2acc.at[pl.ds(r0, CH)],
                                  in_sem0).wait()
        plsc.subcore_barrier()

        cn = c * N
        ebase = e0 + s * eps
        lbase = s * eps
        src_v = (src_v0, src_v1)
        dst_v = (dst_v0, dst_v1)
        dsta_v = (dsta_v0, dsta_v1)
        dst_s = (dst_s0, dst_s1)
        a_buf = (a_buf0, a_buf1)
        b_buf = (b_buf0, b_buf1)
        efw_buf = (efw_buf0, efw_buf1)
        upd_buf = (upd_buf0, upd_buf1)
        gate_buf = (gate_buf0, gate_buf1)
        idx_sem = (idx_sem0, idx_sem1)
        in_sem = (in_sem0, in_sem1)
        out_sem = (out_sem0, out_sem1)

        def issue_idx(j, p):
            jb = ebase + j * CH
            pltpu.async_copy(src.at[pl.ds(jb, CH)], src_v[p], idx_sem[p])
            pltpu.async_copy(dst.at[pl.ds(jb, CH)], dst_v[p], idx_sem[p])

        def wait_idx(p):
            pltpu.make_async_copy(src.at[pl.ds(0, CH)], src_v[p],
                                  idx_sem[p]).wait()
            pltpu.make_async_copy(dst.at[pl.ds(0, CH)], dst_v[p],
                                  idx_sem[p]).wait()

        def adjust(p):
            for k in range(CH // 16):
                sl = pl.ds(k * 16, 16)
                src_v[p][sl] = src_v[p][sl] + cn
                dsta_v[p][sl] = dst_v[p][sl] + cn

        def issue_in(j, p):
            pltpu.async_copy(a2.at[src_v[p]], a_buf[p], in_sem[p])
            pltpu.async_copy(b2.at[dsta_v[p]], b_buf[p], in_sem[p])
            pltpu.async_copy(efw2.at[pl.ds(c * eh + lbase + j * CH, CH)],
                             efw_buf[p], in_sem[p])

        def wait_in(p):
            pltpu.make_async_copy(a2.at[src_v[p]], a_buf[p], in_sem[p]).wait()
            pltpu.make_async_copy(b2.at[dsta_v[p]], b_buf[p], in_sem[p]).wait()
            pltpu.make_async_copy(efw2.at[pl.ds(0, CH)], efw_buf[p],
                                  in_sem[p]).wait()

        def drain_out(p):
            pltpu.make_async_copy(upd_buf[p], upd_o.at[pl.ds(0, CH)],
                                  out_sem[p]).wait()

        def compute(p, bn):
            ab, bb, eb, ub, gb = a_buf[p], b_buf[p], efw_buf[p], upd_buf[p], \
                gate_buf[p]

            def row(r, bn_c):
                out = list(bn_c)
                for j in range(4):
                    sl = pl.ds(j * 16, 16)
                    u = eb[r, sl] + ab[r, sl] + bb[r, sl]
                    ub[r, sl] = u
                    g = u / (1.0 + jnp.exp(-u))
                    gb[r, sl] = g
                    out[j] = bn_c[j] + u
                    out[4 + j] = bn_c[4 + j] + u * u
                return tuple(out)

            return lax.fori_loop(0, CH, row, bn)

        def sub(t, j, p, q, guard_next, bn):
            @pl.when(t >= 1)
            def _():
                drain_out(p)

            wait_in(p)
            for k in range(CH // 16):
                sl = pl.ds(k * 16, 16)
                dst_s[p][sl] = dst_v[p][sl]

            @pl.when(t <= nchunk // 2 - 2)
            def _():
                issue_idx(j + 2, p)

            def start_next():
                wait_idx(q)
                adjust(q)
                issue_in(j + 1, q)

            if guard_next:
                @pl.when(t <= nchunk // 2 - 2)
                def _():
                    start_next()
            else:
                start_next()

            bn = compute(p, bn)
            pltpu.async_copy(upd_buf[p],
                             upd_o.at[pl.ds(c * eh + lbase + j * CH, CH)],
                             out_sem[p])
            pltpu.sync_copy(gate_buf[p], s2acc.at[dst_s[p]], add=True)
            return bn

        issue_idx(0, 0)
        issue_idx(1, 1)
        wait_idx(0)
        adjust(0)
        issue_in(0, 0)

        zero16 = jnp.zeros((16,), jnp.float32)

        def body(t, bn):
            bn = sub(t, 2 * t, 0, 1, False, bn)
            bn = sub(t, 2 * t + 1, 1, 0, True, bn)
            return bn

        bn = lax.fori_loop(0, nchunk // 2, body, (zero16,) * 8)
        drain_out(0)
        drain_out(1)

        for j in range(4):
            sl = pl.ds(j * 16, 16)
            bn_buf[0, sl] = bn[j]
            bn_buf[1, sl] = bn[4 + j]
            for r in range(2, 8):
                bn_buf[r, sl] = zero16
        pltpu.sync_copy(bn_buf, stage.at[s])
        plsc.subcore_barrier()

        ca = c * NACC
        pltpu.sync_copy(s2acc.at[pl.ds(r0, RPS)], s2_o.at[pl.ds(ca + r0, RPS)])

        @pl.when(s == 0)
        def _():
            pltpu.sync_copy(stage, red_buf)
            for j in range(8):
                row_i = j // 4
                sl = pl.ds((j % 4) * 16, 16)
                acc = red_buf[0, row_i, sl]
                for t in range(1, NSUB):
                    acc = acc + red_buf[t, row_i, sl]
                bn_buf[row_i, sl] = acc
            pltpu.sync_copy(bn_buf, bns_o.at[c])

    return _sc_body


def _sc_half(a2, b2, efw2, src, dst, eh, e0):
    mesh = plsc.VectorSubcoreMesh(core_axis_name="c", subcore_axis_name="s")
    fn = pl.kernel(
        _make_sc_body(eh, e0),
        out_type=[
            jax.ShapeDtypeStruct((2 * eh, H), jnp.float32),
            jax.ShapeDtypeStruct((2 * NACC, H), jnp.float32),
            jax.ShapeDtypeStruct((2, 8, H), jnp.float32),
        ],
        mesh=mesh,
        scratch_types=(
            [pltpu.VMEM((CH,), jnp.int32)] * 8
            + [pltpu.VMEM((CH, H), jnp.float32)] * 10
            + [
                pltpu.VMEM((8, H), jnp.float32),
                pltpu.VMEM((NSUB, 8, H), jnp.float32),
                pltpu.VMEM_SHARED((NACC, H), jnp.float32),
                pltpu.VMEM_SHARED((NSUB, 8, H), jnp.float32),
            ]
            + [pltpu.SemaphoreType.DMA] * 6
        ),
        compiler_params=pltpu.CompilerParams(use_tc_tiling_on_sc=False),
    )
    return fn(a2, b2, efw2, src, dst)


def _silu(x):
    return x / (1.0 + jnp.exp(-x))


def _edge_post_body(upd, ef, bns_a, bns_b, gamma, beta, *rest):
    o = rest[-1]
    u = jnp.concatenate([upd[0], upd[1]], axis=1)
    st = bns_a[...] + bns_b[...]
    mean = jnp.concatenate([st[0, 0:1, :], st[1, 0:1, :]], axis=1) * (1.0 / E)
    msq = jnp.concatenate([st[0, 1:2, :], st[1, 1:2, :]], axis=1) * (1.0 / E)
    var = msq - mean * mean
    xh = (u - mean) / jnp.sqrt(var + 1e-5) * gamma[...] + beta[...]
    o[...] = _silu(xh) + ef[...]


def _edge_post(upd3, ef, bns_a, bns_b, gamma, beta, eh, off, prev=None):
    in_specs = [
        pl.BlockSpec((2, EB, H), lambda i: (0, i, 0)),
        pl.BlockSpec((EB, D), lambda i: (i + off, 0)),
        pl.BlockSpec((2, 8, H), lambda i: (0, 0, 0)),
        pl.BlockSpec((2, 8, H), lambda i: (0, 0, 0)),
        pl.BlockSpec((1, D), lambda i: (0, 0)),
        pl.BlockSpec((1, D), lambda i: (0, 0)),
    ]
    args = [upd3, ef, bns_a, bns_b, gamma, beta]
    aliases = {}
    if prev is not None:
        in_specs.append(pl.BlockSpec(memory_space=pltpu.MemorySpace.HBM))
        args.append(prev)
        aliases = {6: 0}
    return pl.pallas_call(
        _edge_post_body,
        grid=(eh // EB,),
        in_specs=in_specs,
        out_specs=pl.BlockSpec((EB, D), lambda i: (i + off, 0)),
        out_shape=jax.ShapeDtypeStruct((E, D), jnp.float32),
        input_output_aliases=aliases,
    )(*args)


def _node_post_body(ns, nd, s2a, s2b, nf, gamma, beta, o, ssum, ssq):
    p = pl.program_id(0)
    i = pl.program_id(1)
    g2 = jnp.concatenate([s2a[0] + s2b[0], s2a[1] + s2b[1]], axis=1)
    upd = ns[...] + nd[...] * (g2 / (g2 + 1e-6))

    @pl.when(p == 0)
    def _():
        @pl.when(i == 0)
        def _():
            ssum[...] = jnp.zeros_like(ssum)
            ssq[...] = jnp.zeros_like(ssq)

        ssum[...] += jnp.sum(upd, axis=0, keepdims=True)
        ssq[...] += jnp.sum(upd * upd, axis=0, keepdims=True)
        o[...] = jnp.zeros_like(o)

    @pl.when(p == 1)
    def _():
        mean = ssum[...] * (1.0 / N)
        var = ssq[...] * (1.0 / N) - mean * mean
        xh = (upd - mean) / jnp.sqrt(var + 1e-5) * gamma[...] + beta[...]
        o[...] = _silu(xh) + nf[...]


def _node_post(ns, nd, s2a, s2b, nf, gamma, beta):
    return pl.pallas_call(
        _node_post_body,
        grid=(2, N // NB),
        in_specs=[
            pl.BlockSpec((NB, D), lambda p, i: (i, 0)),
            pl.BlockSpec((NB, D), lambda p, i: (i, 0)),
            pl.BlockSpec((2, NB, H), lambda p, i: (0, i, 0)),
            pl.BlockSpec((2, NB, H), lambda p, i: (0, i, 0)),
            pl.BlockSpec((NB, D), lambda p, i: (i, 0)),
            pl.BlockSpec((1, D), lambda p, i: (0, 0)),
            pl.BlockSpec((1, D), lambda p, i: (0, 0)),
        ],
        out_specs=pl.BlockSpec((NB, D), lambda p, i: (i, 0)),
        out_shape=jax.ShapeDtypeStruct((N, D), jnp.float32),
        scratch_shapes=[
            pltpu.VMEM((1, D), jnp.float32),
            pltpu.VMEM((1, D), jnp.float32),
        ],
    )(ns, nd, s2a, s2b, nf, gamma, beta)


def kernel(node_feats, edge_feats, W_esrc, b_esrc, W_edst, b_edst, W_eedge,
           b_eedge, W_nsrc, b_nsrc, W_ndst, b_ndst, gamma_e, beta_e, gamma_n,
           beta_n, edge_index):
    src = edge_index[0]
    dst = edge_index[1]
    a3, b3, nd, ns = _prep(node_feats,
                           W_esrc, b_esrc.reshape(1, D),
                           W_edst, b_edst.reshape(1, D),
                           W_ndst, b_ndst.reshape(1, D),
                           W_nsrc, b_nsrc.reshape(1, D))
    a2 = a3.reshape(2 * N, H)
    b2 = b3.reshape(2 * N, H)
    ge = gamma_e.reshape(1, D)
    be = beta_e.reshape(1, D)

    efw_a = _edge_mm(edge_feats, W_eedge, b_eedge.reshape(1, D), E_A, 0)
    upd_a, s2a, bns_a = _sc_half(a2, b2, efw_a.reshape(2 * E_A, H),
                                 src, dst, E_A, 0)
    efw_b = _edge_mm(edge_feats, W_eedge, b_eedge.reshape(1, D), E_B,
                     E_A // EB)
    upd_b, s2b, bns_b = _sc_half(a2, b2, efw_b.reshape(2 * E_B, H),
                                 src, dst, E_B, E_A)
    out_a = _edge_post(upd_a.reshape(2, E_A, H), edge_feats, bns_a, bns_b,
                       ge, be, E_A, 0)
    edge_out = _edge_post(upd_b.reshape(2, E_B, H), edge_feats, bns_a, bns_b,
                          ge, be, E_B, E_A // EB, prev=out_a)
    node_out = _node_post(ns, nd, s2a.reshape(2, NACC, H),
                          s2b.reshape(2, NACC, H), node_feats,
                          gamma_n.reshape(1, D), beta_n.reshape(1, D))
    return (node_out, edge_out)
